# baseline (speedup 1.0000x reference)
import functools

import jax
import jax.numpy as jnp
from jax import lax
from jax.experimental import pallas as pl
from jax.experimental.pallas import tpu as pltpu
from jax.experimental.pallas import tpu_sc as plsc

B = 8
N = 2048
K = 20
OC = 128
RB = 1024
NW = 32
PW = (B * N) // NW
SB = 16
NSB = PW // SB
NST = SB * K // 80
MCOUNT = float(B * N * K)
GRP = OC // 16


def _topk_iter(d, b):
    lane = lax.broadcasted_iota(jnp.int32, d.shape, 1)
    neg_inf = jnp.float32(-jnp.inf)
    vals = d
    cols = []
    for _ in range(K):
        m = jnp.max(vals, axis=1, keepdims=True)
        am = jnp.min(jnp.where(vals == m, lane, N), axis=1, keepdims=True)
        cols.append(am + b * N)
        vals = jnp.where(lane == am, neg_inf, vals)
    return jnp.concatenate(cols, axis=1)


def _pair_dist(xr, xa):
    inner = -2.0 * lax.dot_general(xr, xa, (((1,), (1,)), ((), ())))
    xx_r = jnp.sum(xr * xr, axis=1, keepdims=True)
    xx_a = jnp.sum(xa * xa, axis=1)[None, :]
    return (-xx_r - inner) - xx_a


def _topk_s1_body(xrows_ref, xall_ref, xr_ref, wb_ref, idx_ref, hc_ref):
    b = pl.program_id(0)
    d = _pair_dist(xrows_ref[0], xall_ref[0])
    idx_ref[0] = _topk_iter(d, b)
    hc_ref[0] = lax.dot_general(xr_ref[0], wb_ref[...],
                                (((1,), (1,)), ((), ())))


def _topk_s1(xp, xr, wb):
    grid = (B, N // RB)
    return pl.pallas_call(
        _topk_s1_body,
        grid=grid,
        in_specs=[
            pl.BlockSpec((1, RB, 8), lambda b, r: (b, r, 0)),
            pl.BlockSpec((1, N, 8), lambda b, r: (b, 0, 0)),
            pl.BlockSpec((1, RB, 8), lambda b, r: (b, r, 0)),
            pl.BlockSpec((OC, 8), lambda b, r: (0, 0)),
        ],
        out_specs=[
            pl.BlockSpec((1, RB, K), lambda b, r: (b, r, 0)),
            pl.BlockSpec((1, RB, OC), lambda b, r: (b, r, 0)),
        ],
        out_shape=[
            jax.ShapeDtypeStruct((B, N, K), jnp.int32),
            jax.ShapeDtypeStruct((B, N, OC), jnp.float32),
        ],
    )(xp, xp, xr, wb)


def _topk_s2_body(xrows_ref, xall_ref, wa_ref, wc_ref, idx_ref, a_ref, c_ref):
    b = pl.program_id(0)
    xr = xrows_ref[0]
    d = _pair_dist(xr, xall_ref[0])
    idx_ref[0] = _topk_iter(d, b)
    a_ref[0] = lax.dot_general(xr, wa_ref[...], (((1,), (1,)), ((), ())))
    c_ref[0] = lax.dot_general(xr, wc_ref[...], (((1,), (1,)), ((), ())))


def _topk_s2(xp, wa, wc):
    grid = (B, N // RB)
    return pl.pallas_call(
        _topk_s2_body,
        grid=grid,
        in_specs=[
            pl.BlockSpec((1, RB, OC), lambda b, r: (b, r, 0)),
            pl.BlockSpec((1, N, OC), lambda b, r: (b, 0, 0)),
            pl.BlockSpec((OC, OC), lambda b, r: (0, 0)),
            pl.BlockSpec((OC, OC), lambda b, r: (0, 0)),
        ],
        out_specs=[
            pl.BlockSpec((1, RB, K), lambda b, r: (b, r, 0)),
            pl.BlockSpec((1, RB, OC), lambda b, r: (b, r, 0)),
            pl.BlockSpec((1, RB, OC), lambda b, r: (b, r, 0)),
        ],
        out_shape=[
            jax.ShapeDtypeStruct((B, N, K), jnp.int32),
            jax.ShapeDtypeStruct((B, N, OC), jnp.float32),
            jax.ShapeDtypeStruct((B, N, OC), jnp.float32),
        ],
    )(xp, xp, wa, wc)


def _bf16_round(v):
    bits = lax.bitcast_convert_type(v, jnp.int32)
    r = bits + jnp.int32(0x7FFF) + ((bits >> 16) & 1)
    return lax.bitcast_convert_type(r & jnp.int32(-65536), jnp.float32)


def _fire(tab_hbm, idx_v, sb, rows_buf, sem):
    for c in range(NST):
        pltpu.async_copy(
            tab_hbm.at[idx_v.at[pl.ds(sb * SB * K + c * 80, 80)]],
            rows_buf.at[pl.ds(c * 80, 80)], sem)


def _drain(tab_hbm, rows_buf, sem):
    pltpu.make_async_copy(tab_hbm.at[pl.ds(0, SB * K)], rows_buf, sem).wait()


def _sc_edge1_body(xyz_hbm, hc_hbm, idx_hbm, wa_hbm, m_hbm, s_hbm, q_hbm,
                   idx_v, rows0_v, rows1_v, ctr_v, hc_v, wa_v, mout_v,
                   sacc_v, qacc_v, sem0, sem1):
    wid = lax.axis_index("s") * 2 + lax.axis_index("c")
    base_pt = wid * PW
    pltpu.sync_copy(idx_hbm.at[pl.ds(base_pt * K, PW * K)], idx_v)
    pltpu.sync_copy(wa_hbm, wa_v)
    for g in range(GRP):
        sl = pl.ds(g * 16, 16)
        sacc_v[sl] = jnp.zeros((16,), jnp.float32)
        qacc_v[sl] = jnp.zeros((16,), jnp.float32)

    def compute(sb, rows_v):
        pt0 = sb * SB
        pltpu.sync_copy(xyz_hbm.at[pl.ds(base_pt + pt0, SB)], ctr_v)
        pltpu.sync_copy(hc_hbm.at[pl.ds(base_pt + pt0, SB)], hc_v)

        def pbody(p, carry):
            ctr16 = ctr_v[p]

            def edge(j):
                d = _bf16_round(rows_v[p * K + j] - ctr16)
                d0 = jnp.broadcast_to(d[0], (16,))
                d1 = jnp.broadcast_to(d[1], (16,))
                d2 = jnp.broadcast_to(d[2], (16,))
                hs = []
                for g in range(GRP):
                    sl = pl.ds(g * 16, 16)
                    h = (d0 * wa_v[0, sl] + d1 * wa_v[1, sl]
                         + d2 * wa_v[2, sl] + hc_v[p, sl])
                    hs.append(h)
                return hs

            acc = []
            h0 = edge(0)
            for g in range(GRP):
                acc.extend((h0[g], h0[g], h0[g] * h0[g]))
            for j in range(1, K):
                hs = edge(j)
                for g in range(GRP):
                    m, s, q = acc[3 * g], acc[3 * g + 1], acc[3 * g + 2]
                    h = hs[g]
                    acc[3 * g] = jnp.maximum(m, h)
                    acc[3 * g + 1] = s + h
                    acc[3 * g + 2] = q + h * h
            for g in range(GRP):
                sl = pl.ds(g * 16, 16)
                mout_v[p, sl] = acc[3 * g]
                sacc_v[sl] = sacc_v[sl] + acc[3 * g + 1]
                qacc_v[sl] = qacc_v[sl] + acc[3 * g + 2]
            return carry

        lax.fori_loop(0, SB, pbody, 0)
        pltpu.sync_copy(mout_v, m_hbm.at[pl.ds(base_pt + pt0, SB)])

    _fire(xyz_hbm, idx_v, 0, rows0_v, sem0)

    def outer(o, carry):
        sb0 = 2 * o
        _fire(xyz_hbm, idx_v, sb0 + 1, rows1_v, sem1)
        _drain(xyz_hbm, rows0_v, sem0)
        compute(sb0, rows0_v)

        @pl.when(o < NSB // 2 - 1)
        def _():
            _fire(xyz_hbm, idx_v, sb0 + 2, rows0_v, sem0)

        _drain(xyz_hbm, rows1_v, sem1)
        compute(sb0 + 1, rows1_v)
        return carry

    lax.fori_loop(0, NSB // 2, outer, 0)
    pltpu.sync_copy(sacc_v, s_hbm.at[wid])
    pltpu.sync_copy(qacc_v, q_hbm.at[wid])


_sc_edge1 = functools.partial(
    pl.kernel,
    mesh=plsc.VectorSubcoreMesh(core_axis_name="c", subcore_axis_name="s"),
    compiler_params=pltpu.CompilerParams(use_tc_tiling_on_sc=False),
    out_type=(
        jax.ShapeDtypeStruct((B * N, OC), jnp.float32),
        jax.ShapeDtypeStruct((NW, OC), jnp.float32),
        jax.ShapeDtypeStruct((NW, OC), jnp.float32),
    ),
    scratch_types=[
        pltpu.VMEM((PW * K,), jnp.int32),
        pltpu.VMEM((SB * K, 16), jnp.float32),
        pltpu.VMEM((SB * K, 16), jnp.float32),
        pltpu.VMEM((SB, 16), jnp.float32),
        pltpu.VMEM((SB, OC), jnp.float32),
        pltpu.VMEM((3, OC), jnp.float32),
        pltpu.VMEM((SB, OC), jnp.float32),
        pltpu.VMEM((OC,), jnp.float32),
        pltpu.VMEM((OC,), jnp.float32),
        pltpu.SemaphoreType.DMA,
        pltpu.SemaphoreType.DMA,
    ],
)(_sc_edge1_body)


def _sc_edge2_body(a_hbm, ct_hbm, idx_hbm, m_hbm, s_hbm, q_hbm,
                   idx_v, rows0_v, rows1_v, ct_v, mout_v, sacc_v, qacc_v,
                   sem0, sem1):
    wid = lax.axis_index("s") * 2 + lax.axis_index("c")
    base_pt = wid * PW
    pltpu.sync_copy(idx_hbm.at[pl.ds(base_pt * K, PW * K)], idx_v)
    for g in range(GRP):
        sl = pl.ds(g * 16, 16)
        sacc_v[sl] = jnp.zeros((16,), jnp.float32)
        qacc_v[sl] = jnp.zeros((16,), jnp.float32)

    def compute(sb, rows_v):
        pt0 = sb * SB
        pltpu.sync_copy(ct_hbm.at[pl.ds(base_pt + pt0, SB)], ct_v)

        def pbody(p, carry):
            for g in range(GRP):
                sl = pl.ds(g * 16, 16)
                cvec = ct_v[p, sl]
                v0 = rows_v[p * K, sl] + cvec
                m, s, q = v0, v0, v0 * v0
                for j in range(1, K):
                    v = rows_v[p * K + j, sl] + cvec
                    m = jnp.maximum(m, v)
                    s = s + v
                    q = q + v * v
                mout_v[p, sl] = m
                sacc_v[sl] = sacc_v[sl] + s
                qacc_v[sl] = qacc_v[sl] + q
            return carry

        lax.fori_loop(0, SB, pbody, 0)
        pltpu.sync_copy(mout_v, m_hbm.at[pl.ds(base_pt + pt0, SB)])

    _fire(a_hbm, idx_v, 0, rows0_v, sem0)

    def outer(o, carry):
        sb0 = 2 * o
        _fire(a_hbm, idx_v, sb0 + 1, rows1_v, sem1)
        _drain(a_hbm, rows0_v, sem0)
        compute(sb0, rows0_v)

        @pl.when(o < NSB // 2 - 1)
        def _():
            _fire(a_hbm, idx_v, sb0 + 2, rows0_v, sem0)

        _drain(a_hbm, rows1_v, sem1)
        compute(sb0 + 1, rows1_v)
        return carry

    lax.fori_loop(0, NSB // 2, outer, 0)
    pltpu.sync_copy(sacc_v, s_hbm.at[wid])
    pltpu.sync_copy(qacc_v, q_hbm.at[wid])


_sc_edge2 = functools.partial(
    pl.kernel,
    mesh=plsc.VectorSubcoreMesh(core_axis_name="c", subcore_axis_name="s"),
    out_type=(
        jax.ShapeDtypeStruct((B * N, OC), jnp.float32),
        jax.ShapeDtypeStruct((NW, OC), jnp.float32),
        jax.ShapeDtypeStruct((NW, OC), jnp.float32),
    ),
    scratch_types=[
        pltpu.VMEM((PW * K,), jnp.int32),
        pltpu.VMEM((SB * K, OC), jnp.float32),
        pltpu.VMEM((SB * K, OC), jnp.float32),
        pltpu.VMEM((SB, OC), jnp.float32),
        pltpu.VMEM((SB, OC), jnp.float32),
        pltpu.VMEM((OC,), jnp.float32),
        pltpu.VMEM((OC,), jnp.float32),
        pltpu.SemaphoreType.DMA,
        pltpu.SemaphoreType.DMA,
    ],
)(_sc_edge2_body)


def _bn_body(m_ref, s_ref, q_ref, g_ref, b_ref, out_ref, *, transpose):
    ssum = jnp.sum(s_ref[...], axis=0, keepdims=True)
    qsum = jnp.sum(q_ref[...], axis=0, keepdims=True)
    mean = ssum / MCOUNT
    var = qsum / MCOUNT - mean * mean
    std = jnp.sqrt(var + 1e-5)
    y = (m_ref[0] - mean) / std * g_ref[...] + b_ref[...]
    y = jnp.where(y > 0, y, 0.2 * y)
    if transpose:
        out_ref[0] = y.T
    else:
        out_ref[0] = y


def _bn_apply(m, ssum, qsum, g, bb, transpose):
    grid = (B, N // RB)
    if transpose:
        out_spec = pl.BlockSpec((1, OC, RB), lambda b, r: (b, 0, r))
        out_shape = jax.ShapeDtypeStruct((B, OC, N), jnp.float32)
    else:
        out_spec = pl.BlockSpec((1, RB, OC), lambda b, r: (b, r, 0))
        out_shape = jax.ShapeDtypeStruct((B, N, OC), jnp.float32)
    return pl.pallas_call(
        functools.partial(_bn_body, transpose=transpose),
        grid=grid,
        in_specs=[
            pl.BlockSpec((1, RB, OC), lambda b, r: (b, r, 0)),
            pl.BlockSpec((NW, OC), lambda b, r: (0, 0)),
            pl.BlockSpec((NW, OC), lambda b, r: (0, 0)),
            pl.BlockSpec((1, OC), lambda b, r: (0, 0)),
            pl.BlockSpec((1, OC), lambda b, r: (0, 0)),
        ],
        out_specs=out_spec,
        out_shape=out_shape,
    )(m, ssum, qsum, g, bb)


def kernel(x, W1, g1, b1, W2, g2, b2):
    xt = jnp.transpose(x, (0, 2, 1))
    xp = jnp.pad(xt, ((0, 0), (0, 0), (0, 5)))

    def bf(t):
        return t.astype(jnp.bfloat16).astype(jnp.float32)

    xr = bf(xp)
    w1b = bf(jnp.pad(W1[:, 3:], ((0, 0), (0, 5))))
    idx1, hc1 = _topk_s1(xp, xr, w1b)
    xyzp = jnp.pad(xt, ((0, 0), (0, 0), (0, 13)))
    m1, s1, q1 = _sc_edge1(
        xyzp.reshape(B * N, 16),
        hc1.reshape(B * N, OC),
        idx1.reshape(B * N * K),
        bf(jnp.transpose(W1[:, :3])),
    )
    x1 = _bn_apply(m1.reshape(B, N, OC), s1, q1,
                   g1.reshape(1, OC), b1.reshape(1, OC), transpose=False)

    w2a = W2[:, :OC]
    w2c = W2[:, OC:] - W2[:, :OC]
    idx2, a2, c2 = _topk_s2(x1, w2a, w2c)
    m2, s2, q2 = _sc_edge2(
        a2.reshape(B * N, OC),
        c2.reshape(B * N, OC),
        idx2.reshape(B * N * K),
    )
    return _bn_apply(m2.reshape(B, N, OC), s2, q2,
                     g2.reshape(1, OC), b2.reshape(1, OC), transpose=True)


# speedup vs baseline: 1.1920x; 1.1920x over previous
import functools

import jax
import jax.numpy as jnp
from jax import lax
from jax.experimental import pallas as pl
from jax.experimental.pallas import tpu as pltpu
from jax.experimental.pallas import tpu_sc as plsc

B = 8
N = 2048
K = 20
OC = 128
RB = 512
NW = 32
PW = (B * N) // NW
SB = 16
NSB = PW // SB
NST = SB * K // 80
MCOUNT = float(B * N * K)
GRP = OC // 16


def _topk_iter(d, b):
    lane = lax.broadcasted_iota(jnp.int32, d.shape, 1)
    neg_inf = jnp.float32(-jnp.inf)
    vals = d
    cols = []
    for _ in range(K):
        m = jnp.max(vals, axis=1, keepdims=True)
        am = jnp.min(jnp.where(vals == m, lane, N), axis=1, keepdims=True)
        cols.append(am + b * N)
        vals = jnp.where(lane == am, neg_inf, vals)
    return jnp.concatenate(cols, axis=1)


def _pair_dist(xr, xa):
    inner = -2.0 * lax.dot_general(xr, xa, (((1,), (1,)), ((), ())))
    xx_r = jnp.sum(xr * xr, axis=1, keepdims=True)
    xx_a = jnp.sum(xa * xa, axis=1)[None, :]
    return (-xx_r - inner) - xx_a


def _topk_s1_body(xrows_ref, xall_ref, xr_ref, wb_ref, idx_ref, hc_ref):
    b = pl.program_id(0)
    d = _pair_dist(xrows_ref[0], xall_ref[0])
    idx_ref[0] = _topk_iter(d, b)
    hc_ref[0] = lax.dot_general(xr_ref[0], wb_ref[...],
                                (((1,), (1,)), ((), ())))


def _topk_s1(xp, xr, wb):
    grid = (B, N // RB)
    return pl.pallas_call(
        _topk_s1_body,
        grid=grid,
        in_specs=[
            pl.BlockSpec((1, RB, 8), lambda b, r: (b, r, 0)),
            pl.BlockSpec((1, N, 8), lambda b, r: (b, 0, 0)),
            pl.BlockSpec((1, RB, 8), lambda b, r: (b, r, 0)),
            pl.BlockSpec((OC, 8), lambda b, r: (0, 0)),
        ],
        out_specs=[
            pl.BlockSpec((1, RB, K), lambda b, r: (b, r, 0)),
            pl.BlockSpec((1, RB, OC), lambda b, r: (b, r, 0)),
        ],
        out_shape=[
            jax.ShapeDtypeStruct((B, N, K), jnp.int32),
            jax.ShapeDtypeStruct((B, N, OC), jnp.float32),
        ],
    )(xp, xp, xr, wb)


def _topk_s2_body(mrows_ref, mall_ref, s_ref, q_ref, g_ref, bb_ref,
                  wa_ref, wc_ref, idx_ref, a_ref, c_ref):
    b = pl.program_id(0)
    ssum = jnp.sum(s_ref[...], axis=0, keepdims=True)
    qsum = jnp.sum(q_ref[...], axis=0, keepdims=True)
    mean = ssum / MCOUNT
    var = qsum / MCOUNT - mean * mean
    std = jnp.sqrt(var + 1e-5)

    def act(t):
        y = (t - mean) / std * g_ref[...] + bb_ref[...]
        return jnp.where(y > 0, y, 0.2 * y)

    xr = act(mrows_ref[0])
    xa = act(mall_ref[0])
    d = _pair_dist(xr, xa)
    idx_ref[0] = _topk_iter(d, b)
    a_ref[0] = lax.dot_general(xr, wa_ref[...], (((1,), (1,)), ((), ())))
    c_ref[0] = lax.dot_general(xr, wc_ref[...], (((1,), (1,)), ((), ())))


def _topk_s2(m1, s1, q1, g, bb, wa, wc):
    grid = (B, N // RB)
    return pl.pallas_call(
        _topk_s2_body,
        grid=grid,
        in_specs=[
            pl.BlockSpec((1, RB, OC), lambda b, r: (b, r, 0)),
            pl.BlockSpec((1, N, OC), lambda b, r: (b, 0, 0)),
            pl.BlockSpec((NW, OC), lambda b, r: (0, 0)),
            pl.BlockSpec((NW, OC), lambda b, r: (0, 0)),
            pl.BlockSpec((1, OC), lambda b, r: (0, 0)),
            pl.BlockSpec((1, OC), lambda b, r: (0, 0)),
            pl.BlockSpec((OC, OC), lambda b, r: (0, 0)),
            pl.BlockSpec((OC, OC), lambda b, r: (0, 0)),
        ],
        out_specs=[
            pl.BlockSpec((1, RB, K), lambda b, r: (b, r, 0)),
            pl.BlockSpec((1, RB, OC), lambda b, r: (b, r, 0)),
            pl.BlockSpec((1, RB, OC), lambda b, r: (b, r, 0)),
        ],
        out_shape=[
            jax.ShapeDtypeStruct((B, N, K), jnp.int32),
            jax.ShapeDtypeStruct((B, N, OC), jnp.float32),
            jax.ShapeDtypeStruct((B, N, OC), jnp.float32),
        ],
    )(m1, m1, s1, q1, g, bb, wa, wc)


def _bf16_round(v):
    bits = lax.bitcast_convert_type(v, jnp.int32)
    r = bits + jnp.int32(0x7FFF) + ((bits >> 16) & 1)
    return lax.bitcast_convert_type(r & jnp.int32(-65536), jnp.float32)


def _fire(tab_hbm, idx_v, sb, rows_buf, sem):
    for c in range(NST):
        pltpu.async_copy(
            tab_hbm.at[idx_v.at[pl.ds(sb * SB * K + c * 80, 80)]],
            rows_buf.at[pl.ds(c * 80, 80)], sem)


def _drain(tab_hbm, rows_buf, sem):
    pltpu.make_async_copy(tab_hbm.at[pl.ds(0, SB * K)], rows_buf, sem).wait()


def _sc_edge1_body(xyz_hbm, hc_hbm, idx_hbm, wa_hbm, m_hbm, s_hbm, q_hbm,
                   idx_v, rows0_v, rows1_v, ctr_v, hc_v, wa_v, mout_v,
                   sacc_v, qacc_v, sem0, sem1):
    wid = lax.axis_index("s") * 2 + lax.axis_index("c")
    base_pt = wid * PW
    pltpu.sync_copy(idx_hbm.at[pl.ds(base_pt * K, PW * K)], idx_v)
    pltpu.sync_copy(wa_hbm, wa_v)
    for g in range(GRP):
        sl = pl.ds(g * 16, 16)
        sacc_v[sl] = jnp.zeros((16,), jnp.float32)
        qacc_v[sl] = jnp.zeros((16,), jnp.float32)

    def compute(sb, rows_v):
        pt0 = sb * SB
        pltpu.sync_copy(xyz_hbm.at[pl.ds(base_pt + pt0, SB)], ctr_v)
        pltpu.sync_copy(hc_hbm.at[pl.ds(base_pt + pt0, SB)], hc_v)

        def pbody(p, carry):
            ctr16 = ctr_v[p]

            def edge(j):
                d = _bf16_round(rows_v[p * K + j] - ctr16)
                d0 = jnp.broadcast_to(d[0], (16,))
                d1 = jnp.broadcast_to(d[1], (16,))
                d2 = jnp.broadcast_to(d[2], (16,))
                hs = []
                for g in range(GRP):
                    sl = pl.ds(g * 16, 16)
                    h = (d0 * wa_v[0, sl] + d1 * wa_v[1, sl]
                         + d2 * wa_v[2, sl] + hc_v[p, sl])
                    hs.append(h)
                return hs

            acc = []
            h0 = edge(0)
            for g in range(GRP):
                acc.extend((h0[g], h0[g], h0[g] * h0[g]))
            for j in range(1, K):
                hs = edge(j)
                for g in range(GRP):
                    m, s, q = acc[3 * g], acc[3 * g + 1], acc[3 * g + 2]
                    h = hs[g]
                    acc[3 * g] = jnp.maximum(m, h)
                    acc[3 * g + 1] = s + h
                    acc[3 * g + 2] = q + h * h
            for g in range(GRP):
                sl = pl.ds(g * 16, 16)
                mout_v[p, sl] = acc[3 * g]
                sacc_v[sl] = sacc_v[sl] + acc[3 * g + 1]
                qacc_v[sl] = qacc_v[sl] + acc[3 * g + 2]
            return carry

        lax.fori_loop(0, SB, pbody, 0)
        pltpu.sync_copy(mout_v, m_hbm.at[pl.ds(base_pt + pt0, SB)])

    _fire(xyz_hbm, idx_v, 0, rows0_v, sem0)

    def outer(o, carry):
        sb0 = 2 * o
        _fire(xyz_hbm, idx_v, sb0 + 1, rows1_v, sem1)
        _drain(xyz_hbm, rows0_v, sem0)
        compute(sb0, rows0_v)

        @pl.when(o < NSB // 2 - 1)
        def _():
            _fire(xyz_hbm, idx_v, sb0 + 2, rows0_v, sem0)

        _drain(xyz_hbm, rows1_v, sem1)
        compute(sb0 + 1, rows1_v)
        return carry

    lax.fori_loop(0, NSB // 2, outer, 0)
    pltpu.sync_copy(sacc_v, s_hbm.at[wid])
    pltpu.sync_copy(qacc_v, q_hbm.at[wid])


_sc_edge1 = functools.partial(
    pl.kernel,
    mesh=plsc.VectorSubcoreMesh(core_axis_name="c", subcore_axis_name="s"),
    compiler_params=pltpu.CompilerParams(use_tc_tiling_on_sc=False),
    out_type=(
        jax.ShapeDtypeStruct((B * N, OC), jnp.float32),
        jax.ShapeDtypeStruct((NW, OC), jnp.float32),
        jax.ShapeDtypeStruct((NW, OC), jnp.float32),
    ),
    scratch_types=[
        pltpu.VMEM((PW * K,), jnp.int32),
        pltpu.VMEM((SB * K, 16), jnp.float32),
        pltpu.VMEM((SB * K, 16), jnp.float32),
        pltpu.VMEM((SB, 16), jnp.float32),
        pltpu.VMEM((SB, OC), jnp.float32),
        pltpu.VMEM((3, OC), jnp.float32),
        pltpu.VMEM((SB, OC), jnp.float32),
        pltpu.VMEM((OC,), jnp.float32),
        pltpu.VMEM((OC,), jnp.float32),
        pltpu.SemaphoreType.DMA,
        pltpu.SemaphoreType.DMA,
    ],
)(_sc_edge1_body)


def _sc_edge2_body(a_hbm, ct_hbm, idx_hbm, m_hbm, s_hbm, q_hbm,
                   idx_v, rows0_v, rows1_v, ct_v, mout_v, sacc_v, qacc_v,
                   sem0, sem1):
    wid = lax.axis_index("s") * 2 + lax.axis_index("c")
    base_pt = wid * PW
    pltpu.sync_copy(idx_hbm.at[pl.ds(base_pt * K, PW * K)], idx_v)
    for g in range(GRP):
        sl = pl.ds(g * 16, 16)
        sacc_v[sl] = jnp.zeros((16,), jnp.float32)
        qacc_v[sl] = jnp.zeros((16,), jnp.float32)

    def compute(sb, rows_v):
        pt0 = sb * SB
        pltpu.sync_copy(ct_hbm.at[pl.ds(base_pt + pt0, SB)], ct_v)

        def pbody(p, carry):
            for g in range(GRP):
                sl = pl.ds(g * 16, 16)
                cvec = ct_v[p, sl]
                v0 = rows_v[p * K, sl] + cvec
                m, s, q = v0, v0, v0 * v0
                for j in range(1, K):
                    v = rows_v[p * K + j, sl] + cvec
                    m = jnp.maximum(m, v)
                    s = s + v
                    q = q + v * v
                mout_v[p, sl] = m
                sacc_v[sl] = sacc_v[sl] + s
                qacc_v[sl] = qacc_v[sl] + q
            return carry

        lax.fori_loop(0, SB, pbody, 0)
        pltpu.sync_copy(mout_v, m_hbm.at[pl.ds(base_pt + pt0, SB)])

    _fire(a_hbm, idx_v, 0, rows0_v, sem0)

    def outer(o, carry):
        sb0 = 2 * o
        _fire(a_hbm, idx_v, sb0 + 1, rows1_v, sem1)
        _drain(a_hbm, rows0_v, sem0)
        compute(sb0, rows0_v)

        @pl.when(o < NSB // 2 - 1)
        def _():
            _fire(a_hbm, idx_v, sb0 + 2, rows0_v, sem0)

        _drain(a_hbm, rows1_v, sem1)
        compute(sb0 + 1, rows1_v)
        return carry

    lax.fori_loop(0, NSB // 2, outer, 0)
    pltpu.sync_copy(sacc_v, s_hbm.at[wid])
    pltpu.sync_copy(qacc_v, q_hbm.at[wid])


_sc_edge2 = functools.partial(
    pl.kernel,
    mesh=plsc.VectorSubcoreMesh(core_axis_name="c", subcore_axis_name="s"),
    out_type=(
        jax.ShapeDtypeStruct((B * N, OC), jnp.float32),
        jax.ShapeDtypeStruct((NW, OC), jnp.float32),
        jax.ShapeDtypeStruct((NW, OC), jnp.float32),
    ),
    scratch_types=[
        pltpu.VMEM((PW * K,), jnp.int32),
        pltpu.VMEM((SB * K, OC), jnp.float32),
        pltpu.VMEM((SB * K, OC), jnp.float32),
        pltpu.VMEM((SB, OC), jnp.float32),
        pltpu.VMEM((SB, OC), jnp.float32),
        pltpu.VMEM((OC,), jnp.float32),
        pltpu.VMEM((OC,), jnp.float32),
        pltpu.SemaphoreType.DMA,
        pltpu.SemaphoreType.DMA,
    ],
)(_sc_edge2_body)


def _bn_body(m_ref, s_ref, q_ref, g_ref, b_ref, out_ref, *, transpose):
    ssum = jnp.sum(s_ref[...], axis=0, keepdims=True)
    qsum = jnp.sum(q_ref[...], axis=0, keepdims=True)
    mean = ssum / MCOUNT
    var = qsum / MCOUNT - mean * mean
    std = jnp.sqrt(var + 1e-5)
    y = (m_ref[0] - mean) / std * g_ref[...] + b_ref[...]
    y = jnp.where(y > 0, y, 0.2 * y)
    if transpose:
        out_ref[0] = y.T
    else:
        out_ref[0] = y


def _bn_apply(m, ssum, qsum, g, bb, transpose):
    grid = (B, N // RB)
    if transpose:
        out_spec = pl.BlockSpec((1, OC, RB), lambda b, r: (b, 0, r))
        out_shape = jax.ShapeDtypeStruct((B, OC, N), jnp.float32)
    else:
        out_spec = pl.BlockSpec((1, RB, OC), lambda b, r: (b, r, 0))
        out_shape = jax.ShapeDtypeStruct((B, N, OC), jnp.float32)
    return pl.pallas_call(
        functools.partial(_bn_body, transpose=transpose),
        grid=grid,
        in_specs=[
            pl.BlockSpec((1, RB, OC), lambda b, r: (b, r, 0)),
            pl.BlockSpec((NW, OC), lambda b, r: (0, 0)),
            pl.BlockSpec((NW, OC), lambda b, r: (0, 0)),
            pl.BlockSpec((1, OC), lambda b, r: (0, 0)),
            pl.BlockSpec((1, OC), lambda b, r: (0, 0)),
        ],
        out_specs=out_spec,
        out_shape=out_shape,
    )(m, ssum, qsum, g, bb)


def kernel(x, W1, g1, b1, W2, g2, b2):
    xt = jnp.transpose(x, (0, 2, 1))
    xp = jnp.pad(xt, ((0, 0), (0, 0), (0, 5)))

    def bf(t):
        return t.astype(jnp.bfloat16).astype(jnp.float32)

    xr = bf(xp)
    w1b = bf(jnp.pad(W1[:, 3:], ((0, 0), (0, 5))))
    idx1, hc1 = _topk_s1(xp, xr, w1b)
    xyzp = jnp.pad(xt, ((0, 0), (0, 0), (0, 13)))
    m1, s1, q1 = _sc_edge1(
        xyzp.reshape(B * N, 16),
        hc1.reshape(B * N, OC),
        idx1.reshape(B * N * K),
        bf(jnp.transpose(W1[:, :3])),
    )
    w2a = W2[:, :OC]
    w2c = W2[:, OC:] - W2[:, :OC]
    idx2, a2, c2 = _topk_s2(m1.reshape(B, N, OC), s1, q1,
                            g1.reshape(1, OC), b1.reshape(1, OC), w2a, w2c)
    m2, s2, q2 = _sc_edge2(
        a2.reshape(B * N, OC),
        c2.reshape(B * N, OC),
        idx2.reshape(B * N * K),
    )
    return _bn_apply(m2.reshape(B, N, OC), s2, q2,
                     g2.reshape(1, OC), b2.reshape(1, OC), transpose=True)


# speedup vs baseline: 1.3353x; 1.1202x over previous
import functools

import jax
import jax.numpy as jnp
from jax import lax
from jax.experimental import pallas as pl
from jax.experimental.pallas import tpu as pltpu
from jax.experimental.pallas import tpu_sc as plsc

B = 8
N = 2048
K = 20
OC = 128
RB = 512
NW = 32
PW = (B * N) // NW
SB = 16
NSB = PW // SB
NST = SB * K // 80
MCOUNT = float(B * N * K)
GRP = OC // 16


def _topk_iter(d, b):
    lane = lax.broadcasted_iota(jnp.int32, d.shape, 1)
    neg_inf = jnp.float32(-jnp.inf)
    vals = d
    cols = []
    for _ in range(K):
        am = jnp.argmax(vals, axis=1).astype(jnp.int32)[:, None]
        cols.append(am + b * N)
        vals = jnp.where(lane == am, neg_inf, vals)
    return jnp.concatenate(cols, axis=1)


def _pair_dist(xr, xa):
    inner = -2.0 * lax.dot_general(xr, xa, (((1,), (1,)), ((), ())))
    xx_r = jnp.sum(xr * xr, axis=1, keepdims=True)
    xx_a = jnp.sum(xa * xa, axis=1)[None, :]
    return (-xx_r - inner) - xx_a


def _topk_s1_body(xrows_ref, xall_ref, xr_ref, wb_ref, idx_ref, hc_ref):
    b = pl.program_id(0)
    d = _pair_dist(xrows_ref[0], xall_ref[0])
    idx_ref[0] = _topk_iter(d, b)
    hc_ref[0] = lax.dot_general(xr_ref[0], wb_ref[...],
                                (((1,), (1,)), ((), ())))


def _topk_s1(xp, xr, wb):
    grid = (B, N // RB)
    return pl.pallas_call(
        _topk_s1_body,
        grid=grid,
        in_specs=[
            pl.BlockSpec((1, RB, 8), lambda b, r: (b, r, 0)),
            pl.BlockSpec((1, N, 8), lambda b, r: (b, 0, 0)),
            pl.BlockSpec((1, RB, 8), lambda b, r: (b, r, 0)),
            pl.BlockSpec((OC, 8), lambda b, r: (0, 0)),
        ],
        out_specs=[
            pl.BlockSpec((1, RB, K), lambda b, r: (b, r, 0)),
            pl.BlockSpec((1, RB, OC), lambda b, r: (b, r, 0)),
        ],
        out_shape=[
            jax.ShapeDtypeStruct((B, N, K), jnp.int32),
            jax.ShapeDtypeStruct((B, N, OC), jnp.float32),
        ],
    )(xp, xp, xr, wb)


def _topk_s2_body(mrows_ref, mall_ref, s_ref, q_ref, g_ref, bb_ref,
                  wa_ref, wc_ref, idx_ref, a_ref, c_ref):
    b = pl.program_id(0)
    ssum = jnp.sum(s_ref[...], axis=0, keepdims=True)
    qsum = jnp.sum(q_ref[...], axis=0, keepdims=True)
    mean = ssum / MCOUNT
    var = qsum / MCOUNT - mean * mean
    std = jnp.sqrt(var + 1e-5)

    def act(t):
        y = (t - mean) / std * g_ref[...] + bb_ref[...]
        return jnp.where(y > 0, y, 0.2 * y)

    xr = act(mrows_ref[0])
    xa = act(mall_ref[0])
    d = _pair_dist(xr, xa)
    idx_ref[0] = _topk_iter(d, b)
    a_ref[0] = lax.dot_general(xr, wa_ref[...], (((1,), (1,)), ((), ())))
    c_ref[0] = lax.dot_general(xr, wc_ref[...], (((1,), (1,)), ((), ())))


def _topk_s2(m1, s1, q1, g, bb, wa, wc):
    grid = (B, N // RB)
    return pl.pallas_call(
        _topk_s2_body,
        grid=grid,
        in_specs=[
            pl.BlockSpec((1, RB, OC), lambda b, r: (b, r, 0)),
            pl.BlockSpec((1, N, OC), lambda b, r: (b, 0, 0)),
            pl.BlockSpec((NW, OC), lambda b, r: (0, 0)),
            pl.BlockSpec((NW, OC), lambda b, r: (0, 0)),
            pl.BlockSpec((1, OC), lambda b, r: (0, 0)),
            pl.BlockSpec((1, OC), lambda b, r: (0, 0)),
            pl.BlockSpec((OC, OC), lambda b, r: (0, 0)),
            pl.BlockSpec((OC, OC), lambda b, r: (0, 0)),
        ],
        out_specs=[
            pl.BlockSpec((1, RB, K), lambda b, r: (b, r, 0)),
            pl.BlockSpec((1, RB, OC), lambda b, r: (b, r, 0)),
            pl.BlockSpec((1, RB, OC), lambda b, r: (b, r, 0)),
        ],
        out_shape=[
            jax.ShapeDtypeStruct((B, N, K), jnp.int32),
            jax.ShapeDtypeStruct((B, N, OC), jnp.float32),
            jax.ShapeDtypeStruct((B, N, OC), jnp.float32),
        ],
    )(m1, m1, s1, q1, g, bb, wa, wc)


def _bf16_round(v):
    bits = lax.bitcast_convert_type(v, jnp.int32)
    r = bits + jnp.int32(0x7FFF) + ((bits >> 16) & 1)
    return lax.bitcast_convert_type(r & jnp.int32(-65536), jnp.float32)


def _fire(tab_hbm, idx_v, sb, rows_buf, sem):
    for c in range(NST):
        pltpu.async_copy(
            tab_hbm.at[idx_v.at[pl.ds(sb * SB * K + c * 80, 80)]],
            rows_buf.at[pl.ds(c * 80, 80)], sem)


def _drain(tab_hbm, rows_buf, sem):
    pltpu.make_async_copy(tab_hbm.at[pl.ds(0, SB * K)], rows_buf, sem).wait()


def _sc_edge1_body(xyz_hbm, hc_hbm, idx_hbm, wa_hbm, m_hbm, s_hbm, q_hbm,
                   idx_v, rows0_v, rows1_v, ctr_v, hc_v, wa_v, mout_v,
                   sacc_v, qacc_v, sem0, sem1):
    wid = lax.axis_index("s") * 2 + lax.axis_index("c")
    base_pt = wid * PW
    pltpu.sync_copy(idx_hbm.at[pl.ds(base_pt * K, PW * K)], idx_v)
    pltpu.sync_copy(wa_hbm, wa_v)
    for g in range(GRP):
        sl = pl.ds(g * 16, 16)
        sacc_v[sl] = jnp.zeros((16,), jnp.float32)
        qacc_v[sl] = jnp.zeros((16,), jnp.float32)

    def compute(sb, rows_v):
        pt0 = sb * SB
        pltpu.sync_copy(xyz_hbm.at[pl.ds(base_pt + pt0, SB)], ctr_v)
        pltpu.sync_copy(hc_hbm.at[pl.ds(base_pt + pt0, SB)], hc_v)

        def pbody(p, carry):
            ctr16 = ctr_v[p]

            def edge(j):
                d = _bf16_round(rows_v[p * K + j] - ctr16)
                d0 = jnp.broadcast_to(d[0], (16,))
                d1 = jnp.broadcast_to(d[1], (16,))
                d2 = jnp.broadcast_to(d[2], (16,))
                hs = []
                for g in range(GRP):
                    sl = pl.ds(g * 16, 16)
                    h = (d0 * wa_v[0, sl] + d1 * wa_v[1, sl]
                         + d2 * wa_v[2, sl] + hc_v[p, sl])
                    hs.append(h)
                return hs

            acc = []
            h0 = edge(0)
            for g in range(GRP):
                acc.extend((h0[g], h0[g], h0[g] * h0[g]))
            for j in range(1, K):
                hs = edge(j)
                for g in range(GRP):
                    m, s, q = acc[3 * g], acc[3 * g + 1], acc[3 * g + 2]
                    h = hs[g]
                    acc[3 * g] = jnp.maximum(m, h)
                    acc[3 * g + 1] = s + h
                    acc[3 * g + 2] = q + h * h
            for g in range(GRP):
                sl = pl.ds(g * 16, 16)
                mout_v[p, sl] = acc[3 * g]
                sacc_v[sl] = sacc_v[sl] + acc[3 * g + 1]
                qacc_v[sl] = qacc_v[sl] + acc[3 * g + 2]
            return carry

        lax.fori_loop(0, SB, pbody, 0)
        pltpu.sync_copy(mout_v, m_hbm.at[pl.ds(base_pt + pt0, SB)])

    _fire(xyz_hbm, idx_v, 0, rows0_v, sem0)

    def outer(o, carry):
        sb0 = 2 * o
        _fire(xyz_hbm, idx_v, sb0 + 1, rows1_v, sem1)
        _drain(xyz_hbm, rows0_v, sem0)
        compute(sb0, rows0_v)

        @pl.when(o < NSB // 2 - 1)
        def _():
            _fire(xyz_hbm, idx_v, sb0 + 2, rows0_v, sem0)

        _drain(xyz_hbm, rows1_v, sem1)
        compute(sb0 + 1, rows1_v)
        return carry

    lax.fori_loop(0, NSB // 2, outer, 0)
    pltpu.sync_copy(sacc_v, s_hbm.at[wid])
    pltpu.sync_copy(qacc_v, q_hbm.at[wid])


_sc_edge1 = functools.partial(
    pl.kernel,
    mesh=plsc.VectorSubcoreMesh(core_axis_name="c", subcore_axis_name="s"),
    compiler_params=pltpu.CompilerParams(use_tc_tiling_on_sc=False),
    out_type=(
        jax.ShapeDtypeStruct((B * N, OC), jnp.float32),
        jax.ShapeDtypeStruct((NW, OC), jnp.float32),
        jax.ShapeDtypeStruct((NW, OC), jnp.float32),
    ),
    scratch_types=[
        pltpu.VMEM((PW * K,), jnp.int32),
        pltpu.VMEM((SB * K, 16), jnp.float32),
        pltpu.VMEM((SB * K, 16), jnp.float32),
        pltpu.VMEM((SB, 16), jnp.float32),
        pltpu.VMEM((SB, OC), jnp.float32),
        pltpu.VMEM((3, OC), jnp.float32),
        pltpu.VMEM((SB, OC), jnp.float32),
        pltpu.VMEM((OC,), jnp.float32),
        pltpu.VMEM((OC,), jnp.float32),
        pltpu.SemaphoreType.DMA,
        pltpu.SemaphoreType.DMA,
    ],
)(_sc_edge1_body)


def _sc_edge2_body(a_hbm, ct_hbm, idx_hbm, m_hbm, s_hbm, q_hbm,
                   idx_v, rows0_v, rows1_v, ct_v, mout_v, sacc_v, qacc_v,
                   sem0, sem1):
    wid = lax.axis_index("s") * 2 + lax.axis_index("c")
    base_pt = wid * PW
    pltpu.sync_copy(idx_hbm.at[pl.ds(base_pt * K, PW * K)], idx_v)
    for g in range(GRP):
        sl = pl.ds(g * 16, 16)
        sacc_v[sl] = jnp.zeros((16,), jnp.float32)
        qacc_v[sl] = jnp.zeros((16,), jnp.float32)

    def compute(sb, rows_v):
        pt0 = sb * SB
        pltpu.sync_copy(ct_hbm.at[pl.ds(base_pt + pt0, SB)], ct_v)

        def pbody(p, carry):
            for g in range(GRP):
                sl = pl.ds(g * 16, 16)
                cvec = ct_v[p, sl]
                v0 = rows_v[p * K, sl] + cvec
                m, s, q = v0, v0, v0 * v0
                for j in range(1, K):
                    v = rows_v[p * K + j, sl] + cvec
                    m = jnp.maximum(m, v)
                    s = s + v
                    q = q + v * v
                mout_v[p, sl] = m
                sacc_v[sl] = sacc_v[sl] + s
                qacc_v[sl] = qacc_v[sl] + q
            return carry

        lax.fori_loop(0, SB, pbody, 0)
        pltpu.sync_copy(mout_v, m_hbm.at[pl.ds(base_pt + pt0, SB)])

    _fire(a_hbm, idx_v, 0, rows0_v, sem0)

    def outer(o, carry):
        sb0 = 2 * o
        _fire(a_hbm, idx_v, sb0 + 1, rows1_v, sem1)
        _drain(a_hbm, rows0_v, sem0)
        compute(sb0, rows0_v)

        @pl.when(o < NSB // 2 - 1)
        def _():
            _fire(a_hbm, idx_v, sb0 + 2, rows0_v, sem0)

        _drain(a_hbm, rows1_v, sem1)
        compute(sb0 + 1, rows1_v)
        return carry

    lax.fori_loop(0, NSB // 2, outer, 0)
    pltpu.sync_copy(sacc_v, s_hbm.at[wid])
    pltpu.sync_copy(qacc_v, q_hbm.at[wid])


_sc_edge2 = functools.partial(
    pl.kernel,
    mesh=plsc.VectorSubcoreMesh(core_axis_name="c", subcore_axis_name="s"),
    out_type=(
        jax.ShapeDtypeStruct((B * N, OC), jnp.float32),
        jax.ShapeDtypeStruct((NW, OC), jnp.float32),
        jax.ShapeDtypeStruct((NW, OC), jnp.float32),
    ),
    scratch_types=[
        pltpu.VMEM((PW * K,), jnp.int32),
        pltpu.VMEM((SB * K, OC), jnp.float32),
        pltpu.VMEM((SB * K, OC), jnp.float32),
        pltpu.VMEM((SB, OC), jnp.float32),
        pltpu.VMEM((SB, OC), jnp.float32),
        pltpu.VMEM((OC,), jnp.float32),
        pltpu.VMEM((OC,), jnp.float32),
        pltpu.SemaphoreType.DMA,
        pltpu.SemaphoreType.DMA,
    ],
)(_sc_edge2_body)


def _bn_body(m_ref, s_ref, q_ref, g_ref, b_ref, out_ref, *, transpose):
    ssum = jnp.sum(s_ref[...], axis=0, keepdims=True)
    qsum = jnp.sum(q_ref[...], axis=0, keepdims=True)
    mean = ssum / MCOUNT
    var = qsum / MCOUNT - mean * mean
    std = jnp.sqrt(var + 1e-5)
    y = (m_ref[0] - mean) / std * g_ref[...] + b_ref[...]
    y = jnp.where(y > 0, y, 0.2 * y)
    if transpose:
        out_ref[0] = y.T
    else:
        out_ref[0] = y


def _bn_apply(m, ssum, qsum, g, bb, transpose):
    grid = (B, N // RB)
    if transpose:
        out_spec = pl.BlockSpec((1, OC, RB), lambda b, r: (b, 0, r))
        out_shape = jax.ShapeDtypeStruct((B, OC, N), jnp.float32)
    else:
        out_spec = pl.BlockSpec((1, RB, OC), lambda b, r: (b, r, 0))
        out_shape = jax.ShapeDtypeStruct((B, N, OC), jnp.float32)
    return pl.pallas_call(
        functools.partial(_bn_body, transpose=transpose),
        grid=grid,
        in_specs=[
            pl.BlockSpec((1, RB, OC), lambda b, r: (b, r, 0)),
            pl.BlockSpec((NW, OC), lambda b, r: (0, 0)),
            pl.BlockSpec((NW, OC), lambda b, r: (0, 0)),
            pl.BlockSpec((1, OC), lambda b, r: (0, 0)),
            pl.BlockSpec((1, OC), lambda b, r: (0, 0)),
        ],
        out_specs=out_spec,
        out_shape=out_shape,
    )(m, ssum, qsum, g, bb)


def kernel(x, W1, g1, b1, W2, g2, b2):
    xt = jnp.transpose(x, (0, 2, 1))
    xp = jnp.pad(xt, ((0, 0), (0, 0), (0, 5)))

    def bf(t):
        return t.astype(jnp.bfloat16).astype(jnp.float32)

    xr = bf(xp)
    w1b = bf(jnp.pad(W1[:, 3:], ((0, 0), (0, 5))))
    idx1, hc1 = _topk_s1(xp, xr, w1b)
    xyzp = jnp.pad(xt, ((0, 0), (0, 0), (0, 13)))
    m1, s1, q1 = _sc_edge1(
        xyzp.reshape(B * N, 16),
        hc1.reshape(B * N, OC),
        idx1.reshape(B * N * K),
        bf(jnp.transpose(W1[:, :3])),
    )
    w2a = W2[:, :OC]
    w2c = W2[:, OC:] - W2[:, :OC]
    idx2, a2, c2 = _topk_s2(m1.reshape(B, N, OC), s1, q1,
                            g1.reshape(1, OC), b1.reshape(1, OC), w2a, w2c)
    m2, s2, q2 = _sc_edge2(
        a2.reshape(B * N, OC),
        c2.reshape(B * N, OC),
        idx2.reshape(B * N * K),
    )
    return _bn_apply(m2.reshape(B, N, OC), s2, q2,
                     g2.reshape(1, OC), b2.reshape(1, OC), transpose=True)
